# baseline (speedup 1.0000x reference)
import functools

import jax
import jax.numpy as jnp
from jax import lax
from jax.experimental import pallas as pl
from jax.experimental.pallas import tpu as pltpu
from jax.experimental.pallas import tpu_sc as plsc

N = 16384
C = 256
OUT_D = 3 * C

_LANES = 16
_NUM_CORES = 2
_NUM_SUBCORES = 16
_NW = _NUM_CORES * _NUM_SUBCORES
_ROWS_PER_W = N // _NW
_R = 64
_CHUNKS = _ROWS_PER_W // _R


def _tracklet_body(obj_hbm, pe_hbm, id_hbm, ts_hbm, te_hbm, out_hbm,
                   obj_v, pe_v, id_v, out_v, idx_v, sem):
    wid = lax.axis_index("s") * _NUM_CORES + lax.axis_index("c")
    row0 = wid * _ROWS_PER_W

    def chunk_body(ci, _):
        base = row0 + ci * _R
        pltpu.sync_copy(ts_hbm.at[pl.ds(base, _R)], idx_v)
        pltpu.async_copy(te_hbm.at[idx_v], out_v, sem).wait()
        pltpu.sync_copy(obj_hbm.at[pl.ds(base, _R)], obj_v)
        pltpu.sync_copy(pe_hbm.at[pl.ds(base, _R)], pe_v)
        pltpu.sync_copy(id_hbm.at[pl.ds(base, _R)], id_v)

        def row_body(i, _c):
            for s, src in enumerate((obj_v, pe_v, id_v)):
                for j in range(C // _LANES):
                    dcol = s * C + j * _LANES
                    out_v[i, pl.ds(dcol, _LANES)] = (
                        out_v[i, pl.ds(dcol, _LANES)]
                        + src[i, pl.ds(j * _LANES, _LANES)])
            return _c

        lax.fori_loop(0, _R, row_body, 0)
        pltpu.sync_copy(out_v, out_hbm.at[pl.ds(base, _R)])
        return _

    lax.fori_loop(0, _CHUNKS, chunk_body, 0)


@jax.jit
def kernel(obj_embedding, pe_3d, id_embedding, rel_timestep, temporal_embed):
    mesh = plsc.VectorSubcoreMesh(core_axis_name="c", subcore_axis_name="s")
    run = functools.partial(
        pl.kernel,
        mesh=mesh,
        out_type=jax.ShapeDtypeStruct((N, OUT_D), jnp.float32),
        scratch_types=[
            pltpu.VMEM((_R, C), jnp.float32),
            pltpu.VMEM((_R, C), jnp.float32),
            pltpu.VMEM((_R, C), jnp.float32),
            pltpu.VMEM((_R, OUT_D), jnp.float32),
            pltpu.VMEM((_R,), jnp.int32),
            pltpu.SemaphoreType.DMA,
        ],
    )(_tracklet_body)
    return run(obj_embedding, pe_3d, id_embedding, rel_timestep,
               temporal_embed)


# speedup vs baseline: 1.5906x; 1.5906x over previous
import functools

import jax
import jax.numpy as jnp
from jax import lax
from jax.experimental import pallas as pl
from jax.experimental.pallas import tpu as pltpu
from jax.experimental.pallas import tpu_sc as plsc

N = 16384
C = 256
OUT_D = 3 * C

_LANES = 16
_NUM_CORES = 2
_NUM_SUBCORES = 16
_NW = _NUM_CORES * _NUM_SUBCORES
_ROWS_PER_W = N // _NW
_R = 64
_CHUNKS = _ROWS_PER_W // _R


def _tracklet_body(obj_hbm, pe_hbm, id_hbm, ts_hbm, te_hbm, out_hbm,
                   te_v, out_v, idx_v, sem):
    wid = lax.axis_index("s") * _NUM_CORES + lax.axis_index("c")
    row0 = wid * _ROWS_PER_W

    def chunk_body(ci, _):
        base = row0 + ci * _R
        pltpu.sync_copy(ts_hbm.at[pl.ds(base, _R)], idx_v)
        gather = pltpu.async_copy(te_hbm.at[idx_v], te_v, sem)
        pltpu.sync_copy(obj_hbm.at[pl.ds(base, _R)], out_v.at[:, pl.ds(0, C)])
        pltpu.sync_copy(pe_hbm.at[pl.ds(base, _R)], out_v.at[:, pl.ds(C, C)])
        pltpu.sync_copy(id_hbm.at[pl.ds(base, _R)],
                        out_v.at[:, pl.ds(2 * C, C)])
        gather.wait()

        def row_body(i, _c):
            for j in range(OUT_D // _LANES):
                col = pl.ds(j * _LANES, _LANES)
                plsc.addupdate(out_v.at[i, col], te_v[i, col])
            return _c

        lax.fori_loop(0, _R, row_body, 0)
        pltpu.sync_copy(out_v, out_hbm.at[pl.ds(base, _R)])
        return _

    lax.fori_loop(0, _CHUNKS, chunk_body, 0)


@jax.jit
def kernel(obj_embedding, pe_3d, id_embedding, rel_timestep, temporal_embed):
    mesh = plsc.VectorSubcoreMesh(core_axis_name="c", subcore_axis_name="s")
    run = functools.partial(
        pl.kernel,
        mesh=mesh,
        out_type=jax.ShapeDtypeStruct((N, OUT_D), jnp.float32),
        scratch_types=[
            pltpu.VMEM((_R, OUT_D), jnp.float32),
            pltpu.VMEM((_R, OUT_D), jnp.float32),
            pltpu.VMEM((_R,), jnp.int32),
            pltpu.SemaphoreType.DMA,
        ],
    )(_tracklet_body)
    return run(obj_embedding, pe_3d, id_embedding, rel_timestep,
               temporal_embed)


# speedup vs baseline: 1.8551x; 1.1663x over previous
import functools

import jax
import jax.numpy as jnp
from jax import lax
from jax.experimental import pallas as pl
from jax.experimental.pallas import tpu as pltpu
from jax.experimental.pallas import tpu_sc as plsc

N = 16384
C = 256
OUT_D = 3 * C
T = 20

_LANES = 16
_NUM_CORES = 2
_NUM_SUBCORES = 16
_NW = _NUM_CORES * _NUM_SUBCORES
_ROWS_PER_W = N // _NW
_R = 32
_CHUNKS = _ROWS_PER_W // _R
_NBUF = 4
_GROUPS = _R // _LANES


def _tracklet_body(obj_hbm, pe_hbm, id_hbm, ts_hbm, te_hbm, out_hbm,
                   table_v, out_v, idx_v, *sems):
    sem_in = sems[:_NBUF]
    sem_out = sems[_NBUF:]
    wid = lax.axis_index("s") * _NUM_CORES + lax.axis_index("c")
    row0 = wid * _ROWS_PER_W

    pltpu.sync_copy(te_hbm, table_v)
    pltpu.sync_copy(ts_hbm.at[pl.ds(row0, _ROWS_PER_W)], idx_v)

    def issue_in(ci):
        b = ci % _NBUF
        base = row0 + ci * _R
        return [
            pltpu.async_copy(obj_hbm.at[pl.ds(base, _R)],
                             out_v.at[b, :, pl.ds(0, C)], sem_in[b]),
            pltpu.async_copy(pe_hbm.at[pl.ds(base, _R)],
                             out_v.at[b, :, pl.ds(C, C)], sem_in[b]),
            pltpu.async_copy(id_hbm.at[pl.ds(base, _R)],
                             out_v.at[b, :, pl.ds(2 * C, C)], sem_in[b]),
        ]

    lanes = lax.iota(jnp.int32, _LANES)
    zeros = jnp.zeros((_LANES,), jnp.int32)

    in_d = {ci: issue_in(ci) for ci in range(min(_NBUF - 1, _CHUNKS))}
    out_d = {}
    for ci in range(_CHUNKS):
        b = ci % _NBUF
        for d in in_d.pop(ci):
            d.wait()

        def row_body(i, _c, ci=ci, b=b):
            dt_vec = idx_v[pl.ds(ci * _R + (i & -_LANES), _LANES)]
            dt = jnp.sum(jnp.where(lanes == (i & (_LANES - 1)), dt_vec,
                                   zeros))
            for j in range(OUT_D // _LANES):
                col = pl.ds(j * _LANES, _LANES)
                plsc.addupdate(out_v.at[b, i, col], table_v[dt, col])
            return _c

        lax.fori_loop(0, _R, row_body, 0)

        if ci - 1 in out_d:
            out_d.pop(ci - 1).wait()
        if ci + _NBUF - 1 < _CHUNKS:
            in_d[ci + _NBUF - 1] = issue_in(ci + _NBUF - 1)
        out_d[ci] = pltpu.async_copy(
            out_v.at[b], out_hbm.at[pl.ds(row0 + ci * _R, _R)], sem_out[b])
    out_d.pop(_CHUNKS - 1).wait()


@jax.jit
def kernel(obj_embedding, pe_3d, id_embedding, rel_timestep, temporal_embed):
    mesh = plsc.VectorSubcoreMesh(core_axis_name="c", subcore_axis_name="s")
    run = functools.partial(
        pl.kernel,
        mesh=mesh,
        compiler_params=pltpu.CompilerParams(needs_layout_passes=False),
        out_type=jax.ShapeDtypeStruct((N, OUT_D), jnp.float32),
        scratch_types=(
            [
                pltpu.VMEM((T, OUT_D), jnp.float32),
                pltpu.VMEM((_NBUF, _R, OUT_D), jnp.float32),
                pltpu.VMEM((_ROWS_PER_W,), jnp.int32),
            ]
            + [pltpu.SemaphoreType.DMA] * (2 * _NBUF)
        ),
    )(_tracklet_body)
    return run(obj_embedding, pe_3d, id_embedding, rel_timestep,
               temporal_embed)


# speedup vs baseline: 3.4890x; 1.8807x over previous
import functools

import jax
import jax.numpy as jnp
from jax import lax
from jax.experimental import pallas as pl
from jax.experimental.pallas import tpu as pltpu
from jax.experimental.pallas import tpu_sc as plsc

N = 16384
C = 256
OUT_D = 3 * C
T = 20

_LANES = 16
_NUM_CORES = 2
_NUM_SUBCORES = 16
_NW = _NUM_CORES * _NUM_SUBCORES
_ROWS_PER_W = N // _NW
_R = 32
_CHUNKS = _ROWS_PER_W // _R
_NBUF = 4
_GROUPS = _R // _LANES


def _tracklet_body(obj_hbm, pe_hbm, id_hbm, ts_hbm, te_hbm, out_hbm,
                   table_v, out_v, idx_v, *sems):
    sem_in = sems[:_NBUF]
    sem_out = sems[_NBUF:]
    wid = lax.axis_index("s") * _NUM_CORES + lax.axis_index("c")
    row0 = wid * _ROWS_PER_W

    pltpu.sync_copy(te_hbm, table_v)
    pltpu.sync_copy(ts_hbm.at[pl.ds(row0, _ROWS_PER_W)], idx_v)

    def issue_in(ci):
        b = ci % _NBUF
        base = row0 + ci * _R
        return [
            pltpu.async_copy(obj_hbm.at[pl.ds(base, _R)],
                             out_v.at[b, :, pl.ds(0, C)], sem_in[b]),
            pltpu.async_copy(pe_hbm.at[pl.ds(base, _R)],
                             out_v.at[b, :, pl.ds(C, C)], sem_in[b]),
            pltpu.async_copy(id_hbm.at[pl.ds(base, _R)],
                             out_v.at[b, :, pl.ds(2 * C, C)], sem_in[b]),
        ]

    lanes = lax.iota(jnp.int32, _LANES)
    zeros = jnp.zeros((_LANES,), jnp.int32)

    in_d = {ci: issue_in(ci) for ci in range(min(_NBUF - 1, _CHUNKS))}
    out_d = {}
    for ci in range(_CHUNKS):
        b = ci % _NBUF
        for d in in_d.pop(ci):
            d.wait()

        @plsc.parallel_loop(0, _R)
        def _row_loop(i, ci=ci, b=b):
            dt_vec = idx_v[pl.ds(ci * _R + (i & -_LANES), _LANES)]
            dt = jnp.sum(jnp.where(lanes == (i & (_LANES - 1)), dt_vec,
                                   zeros))
            cols = [pl.ds(j * _LANES, _LANES)
                    for j in range(OUT_D // _LANES)]
            te = [table_v[dt, col] for col in cols]
            for col, te_col in zip(cols, te):
                plsc.addupdate(out_v.at[b, i, col], te_col)

        if ci - 1 in out_d:
            out_d.pop(ci - 1).wait()
        if ci + _NBUF - 1 < _CHUNKS:
            in_d[ci + _NBUF - 1] = issue_in(ci + _NBUF - 1)
        out_d[ci] = pltpu.async_copy(
            out_v.at[b], out_hbm.at[pl.ds(row0 + ci * _R, _R)], sem_out[b])
    out_d.pop(_CHUNKS - 1).wait()


@jax.jit
def kernel(obj_embedding, pe_3d, id_embedding, rel_timestep, temporal_embed):
    mesh = plsc.VectorSubcoreMesh(core_axis_name="c", subcore_axis_name="s")
    run = functools.partial(
        pl.kernel,
        mesh=mesh,
        compiler_params=pltpu.CompilerParams(needs_layout_passes=False),
        out_type=jax.ShapeDtypeStruct((N, OUT_D), jnp.float32),
        scratch_types=(
            [
                pltpu.VMEM((T, OUT_D), jnp.float32),
                pltpu.VMEM((_NBUF, _R, OUT_D), jnp.float32),
                pltpu.VMEM((_ROWS_PER_W,), jnp.int32),
            ]
            + [pltpu.SemaphoreType.DMA] * (2 * _NBUF)
        ),
    )(_tracklet_body)
    return run(obj_embedding, pe_3d, id_embedding, rel_timestep,
               temporal_embed)


# speedup vs baseline: 3.8895x; 1.1148x over previous
import functools

import jax
import jax.numpy as jnp
from jax import lax
from jax.experimental import pallas as pl
from jax.experimental.pallas import tpu as pltpu
from jax.experimental.pallas import tpu_sc as plsc

N = 16384
C = 256
OUT_D = 3 * C
T = 20

_LANES = 16
_NUM_CORES = 2
_NUM_SUBCORES = 16
_NW = _NUM_CORES * _NUM_SUBCORES
_ROWS_PER_W = N // _NW
_R = 32
_CHUNKS = _ROWS_PER_W // _R
_NBUF = 4
_GROUPS = _R // _LANES


def _tracklet_body(obj_hbm, pe_hbm, id_hbm, ts_hbm, te_hbm, out_hbm,
                   table_v, out_v, idx_v, *sems):
    sem_in = sems[:_NBUF]
    sem_out = sems[_NBUF:]
    wid = lax.axis_index("s") * _NUM_CORES + lax.axis_index("c")
    row0 = wid * _ROWS_PER_W

    pltpu.sync_copy(te_hbm, table_v)
    pltpu.sync_copy(ts_hbm.at[pl.ds(row0, _ROWS_PER_W)], idx_v)

    strips = ((obj_hbm, 0), (pe_hbm, C), (id_hbm, 2 * C))

    def issue_in(base, b):
        for src, c0 in strips:
            pltpu.async_copy(src.at[pl.ds(base, _R)],
                             out_v.at[b, :, pl.ds(c0, C)], sem_in[b])

    def wait_in(base, b):
        for src, c0 in strips:
            pltpu.make_async_copy(src.at[pl.ds(base, _R)],
                                  out_v.at[b, :, pl.ds(c0, C)],
                                  sem_in[b]).wait()

    lanes = lax.iota(jnp.int32, _LANES)
    zeros = jnp.zeros((_LANES,), jnp.int32)

    for ci in range(_NBUF - 1):
        issue_in(row0 + ci * _R, ci)

    def cycle_body(cg, _):
        for s in range(_NBUF):
            ci = cg * _NBUF + s
            base = row0 + ci * _R
            wait_in(base, s)

            @plsc.parallel_loop(0, _R)
            def _row_loop(i, ci=ci, s=s):
                dt_vec = idx_v[pl.ds(ci * _R + (i & -_LANES), _LANES)]
                dt = jnp.sum(jnp.where(lanes == (i & (_LANES - 1)), dt_vec,
                                       zeros))
                cols = [pl.ds(j * _LANES, _LANES)
                        for j in range(OUT_D // _LANES)]
                te = [table_v[dt, col] for col in cols]
                for col, te_col in zip(cols, te):
                    plsc.addupdate(out_v.at[s, i, col], te_col)

            ps = (s - 1) % _NBUF

            @pl.when(ci >= 1)
            def _wait_prev_out(ps=ps, base=base):
                pltpu.make_async_copy(
                    out_v.at[ps], out_hbm.at[pl.ds(base - _R, _R)],
                    sem_out[ps]).wait()

            @pl.when(ci + _NBUF - 1 < _CHUNKS)
            def _issue_next_in(ps=ps, base=base):
                issue_in(base + (_NBUF - 1) * _R, ps)

            pltpu.async_copy(out_v.at[s], out_hbm.at[pl.ds(base, _R)],
                             sem_out[s])
        return _

    lax.fori_loop(0, _CHUNKS // _NBUF, cycle_body, 0)
    pltpu.make_async_copy(
        out_v.at[_NBUF - 1],
        out_hbm.at[pl.ds(row0 + (_CHUNKS - 1) * _R, _R)],
        sem_out[_NBUF - 1]).wait()


@jax.jit
def kernel(obj_embedding, pe_3d, id_embedding, rel_timestep, temporal_embed):
    mesh = plsc.VectorSubcoreMesh(core_axis_name="c", subcore_axis_name="s")
    run = functools.partial(
        pl.kernel,
        mesh=mesh,
        compiler_params=pltpu.CompilerParams(needs_layout_passes=False),
        out_type=jax.ShapeDtypeStruct((N, OUT_D), jnp.float32),
        scratch_types=(
            [
                pltpu.VMEM((T, OUT_D), jnp.float32),
                pltpu.VMEM((_NBUF, _R, OUT_D), jnp.float32),
                pltpu.VMEM((_ROWS_PER_W,), jnp.int32),
            ]
            + [pltpu.SemaphoreType.DMA] * (2 * _NBUF)
        ),
    )(_tracklet_body)
    return run(obj_embedding, pe_3d, id_embedding, rel_timestep,
               temporal_embed)
